# Initial kernel scaffold; baseline (speedup 1.0000x reference)
import jax
import jax.numpy as jnp
from jax.experimental import pallas as pl


def kernel(x, edge_index, edge_weight, W1, b1, W2, b2):
    raise NotImplementedError("write your pallas kernel here")


# speedup vs baseline: 11.6201x; 11.6201x over previous
import functools

import jax
import jax.numpy as jnp
from jax import lax
from jax.experimental import pallas as pl
from jax.experimental.pallas import tpu as pltpu
from jax.experimental.pallas import tpu_sc as plsc

NC = 2
NS = 16
NW = NC * NS
L = 16
CH = 80

_mesh = lambda: plsc.VectorSubcoreMesh(
    core_axis_name="c", subcore_axis_name="s", num_cores=NC, num_subcores=NS)


@functools.lru_cache(maxsize=None)
def _deg_kernel(E, Npad):
  EPT = E // NW
  NCH = EPT // CH

  @functools.partial(
      pl.kernel, mesh=_mesh(),
      compiler_params=pltpu.CompilerParams(needs_layout_passes=False, use_tc_tiling_on_sc=False),
      out_type=jax.ShapeDtypeStruct((NW, Npad), jnp.float32),
      scratch_types=[
          pltpu.VMEM((Npad,), jnp.float32),
          pltpu.VMEM((CH,), jnp.int32),
          pltpu.VMEM((CH,), jnp.float32),
      ],
  )
  def k(dst_hbm, ew_hbm, out_hbm, acc, dstv, ewv):
    cid = lax.axis_index("c")
    sid = lax.axis_index("s")
    wid = sid * NC + cid
    zv = jnp.zeros((L,), jnp.float32)

    def zero(i, _):
      acc[pl.ds(i * L, L)] = zv
      return 0
    lax.fori_loop(0, Npad // L, zero, 0)

    def chunk(c, _):
      base = wid * EPT + c * CH
      pltpu.sync_copy(dst_hbm.at[pl.ds(base, CH)], dstv)
      pltpu.sync_copy(ew_hbm.at[pl.ds(base, CH)], ewv)
      for kk in range(CH // L):
        idx = dstv[pl.ds(kk * L, L)]
        w = ewv[pl.ds(kk * L, L)]
        plsc.addupdate_scatter(acc, [idx], w)
      return 0
    lax.fori_loop(0, NCH, chunk, 0)
    pltpu.sync_copy(acc, out_hbm.at[wid])

  return k


@functools.lru_cache(maxsize=None)
def _spmm_kernel(E, Npad, D):
  EPT = E // NW
  NCH = EPT // CH
  SEG = Npad // NS
  NZ = SEG // CH

  @functools.partial(
      pl.kernel, mesh=_mesh(),
      compiler_params=pltpu.CompilerParams(needs_layout_passes=False, use_tc_tiling_on_sc=False),
      out_type=jax.ShapeDtypeStruct((NC, Npad, D), jnp.float32),
      scratch_types=[
          pltpu.VMEM_SHARED((Npad, D), jnp.float32),
          pltpu.VMEM((CH, D), jnp.float32),
          pltpu.VMEM((CH,), jnp.int32),
          pltpu.VMEM((CH,), jnp.int32),
          pltpu.VMEM((CH, L), jnp.float32),
          pltpu.VMEM((CH, D), jnp.float32),
          pltpu.SemaphoreType.DMA,
      ],
  )
  def k(tab_hbm, src_hbm, dst_hbm, ewb_hbm, out_hbm,
        acc, rows, srcv, dstv, ewv, zbuf, sem):
    cid = lax.axis_index("c")
    sid = lax.axis_index("s")
    wid = sid * NC + cid
    zv = jnp.zeros((L,), jnp.float32)

    def zfill(i, _):
      for f in range(D // L):
        zbuf[i, pl.ds(f * L, L)] = zv
      return 0
    lax.fori_loop(0, CH, zfill, 0)

    def zcopy(i, _):
      pltpu.sync_copy(zbuf, acc.at[pl.ds(sid * SEG + i * CH, CH)])
      return 0
    lax.fori_loop(0, NZ, zcopy, 0)
    plsc.subcore_barrier()

    def chunk(c, _):
      base = wid * EPT + c * CH
      pltpu.sync_copy(src_hbm.at[pl.ds(base, CH)], srcv)
      pltpu.sync_copy(dst_hbm.at[pl.ds(base, CH)], dstv)
      pltpu.sync_copy(ewb_hbm.at[pl.ds(base, CH)], ewv)
      pltpu.async_copy(tab_hbm.at[srcv], rows, sem).wait()

      def scale(j, _):
        w = ewv[j]
        for f in range(D // L):
          rows[j, pl.ds(f * L, L)] = rows[j, pl.ds(f * L, L)] * w
        return 0
      lax.fori_loop(0, CH, scale, 0)
      pltpu.sync_copy(rows, acc.at[dstv], add=True)
      return 0
    lax.fori_loop(0, NCH, chunk, 0)

    plsc.subcore_barrier()
    pltpu.sync_copy(acc.at[pl.ds(sid * SEG, SEG)],
                    out_hbm.at[cid, pl.ds(sid * SEG, SEG)])

  return k


@functools.lru_cache(maxsize=None)
def _prep_call(N, F, Npad, Rb):
  def body(degp_ref, x_ref, dinv_ref, xs_ref):
    deg = jnp.sum(degp_ref[...], axis=0) + 1.0
    dinv = lax.rsqrt(deg)
    dinv_ref[...] = dinv
    xs_ref[...] = x_ref[...] * dinv[:, None]

  return pl.pallas_call(
      body,
      grid=(Npad // Rb,),
      in_specs=[
          pl.BlockSpec((NW, Rb), lambda i: (0, i)),
          pl.BlockSpec((Rb, F), lambda i: (i, 0)),
      ],
      out_specs=[
          pl.BlockSpec((Rb,), lambda i: (i,)),
          pl.BlockSpec((Rb, F), lambda i: (i, 0)),
      ],
      out_shape=[
          jax.ShapeDtypeStruct((N,), jnp.float32),
          jax.ShapeDtypeStruct((N, F), jnp.float32),
      ],
  )


@functools.lru_cache(maxsize=None)
def _mid_call(N, F, H, C, Npad, Rb):
  def body(aggp_ref, xs_ref, dinv_ref, w1_ref, b1_ref, w2_ref, zs_ref):
    dinv = dinv_ref[...]
    a = (aggp_ref[0] + aggp_ref[1] + xs_ref[...]) * dinv[:, None]
    h = jnp.dot(a, w1_ref[...], preferred_element_type=jnp.float32) + b1_ref[...]
    h = jnp.where(h > 0.0, h, jnp.exp(h) - 1.0)
    z = jnp.dot(h, w2_ref[...], preferred_element_type=jnp.float32)
    zs_ref[...] = z * dinv[:, None]

  return pl.pallas_call(
      body,
      grid=(Npad // Rb,),
      in_specs=[
          pl.BlockSpec((NC, Rb, F), lambda i: (0, i, 0)),
          pl.BlockSpec((Rb, F), lambda i: (i, 0)),
          pl.BlockSpec((Rb,), lambda i: (i,)),
          pl.BlockSpec((F, H), lambda i: (0, 0)),
          pl.BlockSpec((H,), lambda i: (0,)),
          pl.BlockSpec((H, C), lambda i: (0, 0)),
      ],
      out_specs=pl.BlockSpec((Rb, C), lambda i: (i, 0)),
      out_shape=jax.ShapeDtypeStruct((N, C), jnp.float32),
  )


@functools.lru_cache(maxsize=None)
def _final_call(N, C, Npad, Rb):
  def body(agg2p_ref, zs_ref, dinv_ref, b2_ref, out_ref):
    dinv = dinv_ref[...]
    o = (agg2p_ref[0] + agg2p_ref[1] + zs_ref[...]) * dinv[:, None] + b2_ref[...]
    m = jnp.max(o, axis=1, keepdims=True)
    e = jnp.exp(o - m)
    s = jnp.sum(e, axis=1, keepdims=True)
    out_ref[...] = o - m - jnp.log(s)

  return pl.pallas_call(
      body,
      grid=(Npad // Rb,),
      in_specs=[
          pl.BlockSpec((NC, Rb, C), lambda i: (0, i, 0)),
          pl.BlockSpec((Rb, C), lambda i: (i, 0)),
          pl.BlockSpec((Rb,), lambda i: (i,)),
          pl.BlockSpec((C,), lambda i: (0,)),
      ],
      out_specs=pl.BlockSpec((Rb, C), lambda i: (i, 0)),
      out_shape=jax.ShapeDtypeStruct((N, C), jnp.float32),
  )


def kernel(x, edge_index, edge_weight, W1, b1, W2, b2):
  N, F = x.shape
  E = edge_weight.shape[0]
  H = W1.shape[1]
  C = W2.shape[1]
  assert E % (NW * CH) == 0

  seg = ((N + NS - 1) // NS + CH - 1) // CH * CH
  Npad = NS * seg
  Rb = 512
  assert Npad % Rb == 0

  src = edge_index[0]
  dst = edge_index[1]
  ewb = jnp.broadcast_to(edge_weight[:, None], (E, L))

  degp = _deg_kernel(E, Npad)(dst, edge_weight)
  dinv, xs = _prep_call(N, F, Npad, Rb)(degp, x)
  aggp = _spmm_kernel(E, Npad, F)(xs, src, dst, ewb)
  zs = _mid_call(N, F, H, C, Npad, Rb)(aggp, xs, dinv, W1, b1, W2)
  agg2p = _spmm_kernel(E, Npad, C)(zs, src, dst, ewb)
  return _final_call(N, C, Npad, Rb)(agg2p, zs, dinv, b2)


# speedup vs baseline: 27.4619x; 2.3633x over previous
import functools

import jax
import jax.numpy as jnp
from jax import lax
from jax.experimental import pallas as pl
from jax.experimental.pallas import tpu as pltpu
from jax.experimental.pallas import tpu_sc as plsc

NC = 2
NS = 16
NW = NC * NS
L = 16
CH = 40

_mesh = lambda: plsc.VectorSubcoreMesh(
    core_axis_name="c", subcore_axis_name="s", num_cores=NC, num_subcores=NS)


@functools.lru_cache(maxsize=None)
def _deg_kernel(E, Npad):
  EPT = E // NW
  NCH = EPT // CH

  @functools.partial(
      pl.kernel, mesh=_mesh(),
      compiler_params=pltpu.CompilerParams(needs_layout_passes=False, use_tc_tiling_on_sc=False),
      out_type=jax.ShapeDtypeStruct((NW, Npad), jnp.float32),
      scratch_types=[
          pltpu.VMEM((Npad,), jnp.float32),
          pltpu.VMEM((EPT,), jnp.int32),
          pltpu.VMEM((EPT,), jnp.float32),
      ],
  )
  def k(dst_hbm, ew_hbm, out_hbm, acc, dstv, ewv):
    cid = lax.axis_index("c")
    sid = lax.axis_index("s")
    wid = sid * NC + cid
    zv = jnp.zeros((L,), jnp.float32)

    def zero(i, _):
      acc[pl.ds(i * L, L)] = zv
      return 0
    lax.fori_loop(0, Npad // L, zero, 0)

    pltpu.sync_copy(dst_hbm.at[wid], dstv)
    pltpu.sync_copy(ew_hbm.at[wid], ewv)

    def step(i, _):
      idx = dstv[pl.ds(i * L, L)]
      w = ewv[pl.ds(i * L, L)]
      plsc.addupdate_scatter(acc, [idx], w)
      return 0
    lax.fori_loop(0, EPT // L, step, 0)
    pltpu.sync_copy(acc, out_hbm.at[wid])

  return k


NBUF = 2


@functools.lru_cache(maxsize=None)
def _spmm_kernel(E, Npad, D):
  EPT = E // NW
  NCH = EPT // CH
  SEG = Npad // NS
  NZ = SEG // CH

  NO = NCH // NBUF

  @functools.partial(
      pl.kernel, mesh=_mesh(),
      compiler_params=pltpu.CompilerParams(needs_layout_passes=False, use_tc_tiling_on_sc=False),
      out_type=jax.ShapeDtypeStruct((NC, Npad, D), jnp.float32),
      scratch_types=[
          pltpu.VMEM_SHARED((Npad, D), jnp.float32),
          pltpu.VMEM((NBUF, CH, D), jnp.float32),
          pltpu.VMEM((NCH, CH), jnp.int32),
          pltpu.VMEM((NCH, CH), jnp.int32),
          pltpu.VMEM((EPT,), jnp.float32),
          pltpu.SemaphoreType.DMA((NBUF,)),
          pltpu.SemaphoreType.DMA((NBUF,)),
      ],
  )
  def k(tab_hbm, src_hbm, dst_hbm, ew_hbm, out_hbm,
        acc, rows, srcv, dstv, ewv, gsem, ssem):
    cid = lax.axis_index("c")
    sid = lax.axis_index("s")
    wid = sid * NC + cid
    zv = jnp.zeros((L,), jnp.float32)

    pltpu.sync_copy(src_hbm.at[wid], srcv)
    pltpu.sync_copy(dst_hbm.at[wid], dstv)
    pltpu.sync_copy(ew_hbm.at[wid], ewv)

    def zfill(i, _):
      for f in range(D // L):
        rows[0, i, pl.ds(f * L, L)] = zv
      return 0
    lax.fori_loop(0, CH, zfill, 0)

    def zcopy(i, _):
      pltpu.sync_copy(rows.at[0], acc.at[pl.ds(sid * SEG + i * CH, CH)])
      return 0
    lax.fori_loop(0, NZ, zcopy, 0)
    plsc.subcore_barrier()

    def gather_start(c, b):
      pltpu.async_copy(tab_hbm.at[srcv.at[c]], rows.at[b], gsem.at[b])

    for b in range(NBUF):
      gather_start(b, b)

    def outer(kk, _):
      for b in range(NBUF):
        c = kk * NBUF + b
        pltpu.make_async_copy(
            tab_hbm.at[srcv.at[c]], rows.at[b], gsem.at[b]).wait()
        rb = rows.at[b]
        e0 = c * CH

        def scale(j, _):
          w = plsc.load_gather(ewv, [jnp.full((L,), e0, jnp.int32) + j])
          for f in range(D // L):
            rb[j, pl.ds(f * L, L)] = rb[j, pl.ds(f * L, L)] * w
          return 0
        lax.fori_loop(0, CH, scale, 0)
        pltpu.async_copy(rb, acc.at[dstv.at[c]], ssem.at[b], add=True)

      for b in range(NBUF):
        c = kk * NBUF + b
        pltpu.make_async_copy(
            rows.at[b], acc.at[dstv.at[c]], ssem.at[b]).wait()
        nc = c + NBUF

        @pl.when(nc < NCH)
        def _():
          gather_start(nc, b)
      return 0
    lax.fori_loop(0, NO, outer, 0)

    plsc.subcore_barrier()
    pltpu.sync_copy(acc.at[pl.ds(sid * SEG, SEG)],
                    out_hbm.at[cid, pl.ds(sid * SEG, SEG)])

  return k


@functools.lru_cache(maxsize=None)
def _prep_call(N, F, Npad, Rb):
  def body(degp_ref, x_ref, dinv_ref, xs_ref):
    deg = jnp.sum(degp_ref[...], axis=0) + 1.0
    dinv = lax.rsqrt(deg)
    dinv_ref[...] = dinv
    xs_ref[...] = x_ref[...] * dinv[:, None]

  return pl.pallas_call(
      body,
      grid=(Npad // Rb,),
      in_specs=[
          pl.BlockSpec((NW, Rb), lambda i: (0, i)),
          pl.BlockSpec((Rb, F), lambda i: (i, 0)),
      ],
      out_specs=[
          pl.BlockSpec((Rb,), lambda i: (i,)),
          pl.BlockSpec((Rb, F), lambda i: (i, 0)),
      ],
      out_shape=[
          jax.ShapeDtypeStruct((N,), jnp.float32),
          jax.ShapeDtypeStruct((N, F), jnp.float32),
      ],
  )


@functools.lru_cache(maxsize=None)
def _mid_call(N, F, H, C, Npad, Rb):
  def body(aggp_ref, xs_ref, dinv_ref, w1_ref, b1_ref, w2_ref, zs_ref):
    dinv = dinv_ref[...]
    a = (aggp_ref[0] + aggp_ref[1] + xs_ref[...]) * dinv[:, None]
    h = jnp.dot(a, w1_ref[...], preferred_element_type=jnp.float32) + b1_ref[...]
    h = jnp.where(h > 0.0, h, jnp.exp(h) - 1.0)
    z = jnp.dot(h, w2_ref[...], preferred_element_type=jnp.float32)
    zs_ref[...] = z * dinv[:, None]

  return pl.pallas_call(
      body,
      grid=(Npad // Rb,),
      in_specs=[
          pl.BlockSpec((NC, Rb, F), lambda i: (0, i, 0)),
          pl.BlockSpec((Rb, F), lambda i: (i, 0)),
          pl.BlockSpec((Rb,), lambda i: (i,)),
          pl.BlockSpec((F, H), lambda i: (0, 0)),
          pl.BlockSpec((H,), lambda i: (0,)),
          pl.BlockSpec((H, C), lambda i: (0, 0)),
      ],
      out_specs=pl.BlockSpec((Rb, C), lambda i: (i, 0)),
      out_shape=jax.ShapeDtypeStruct((N, C), jnp.float32),
  )


@functools.lru_cache(maxsize=None)
def _final_call(N, C, Npad, Rb):
  def body(agg2p_ref, zs_ref, dinv_ref, b2_ref, out_ref):
    dinv = dinv_ref[...]
    o = (agg2p_ref[0] + agg2p_ref[1] + zs_ref[...]) * dinv[:, None] + b2_ref[...]
    m = jnp.max(o, axis=1, keepdims=True)
    e = jnp.exp(o - m)
    s = jnp.sum(e, axis=1, keepdims=True)
    out_ref[...] = o - m - jnp.log(s)

  return pl.pallas_call(
      body,
      grid=(Npad // Rb,),
      in_specs=[
          pl.BlockSpec((NC, Rb, C), lambda i: (0, i, 0)),
          pl.BlockSpec((Rb, C), lambda i: (i, 0)),
          pl.BlockSpec((Rb,), lambda i: (i,)),
          pl.BlockSpec((C,), lambda i: (0,)),
      ],
      out_specs=pl.BlockSpec((Rb, C), lambda i: (i, 0)),
      out_shape=jax.ShapeDtypeStruct((N, C), jnp.float32),
  )


def kernel(x, edge_index, edge_weight, W1, b1, W2, b2):
  N, F = x.shape
  E = edge_weight.shape[0]
  H = W1.shape[1]
  C = W2.shape[1]
  assert E % (NW * CH) == 0

  seg = ((N + NS - 1) // NS + CH - 1) // CH * CH
  Npad = NS * seg
  Rb = 512
  assert Npad % Rb == 0

  EPT = E // NW
  NCH = EPT // CH
  src3 = edge_index[0].reshape(NW, NCH, CH)
  dst3 = edge_index[1].reshape(NW, NCH, CH)
  dst2 = edge_index[1].reshape(NW, EPT)
  ew2 = edge_weight.reshape(NW, EPT)

  degp = _deg_kernel(E, Npad)(dst2, ew2)
  dinv, xs = _prep_call(N, F, Npad, Rb)(degp, x)
  aggp = _spmm_kernel(E, Npad, F)(xs, src3, dst3, ew2)
  zs = _mid_call(N, F, H, C, Npad, Rb)(aggp, xs, dinv, W1, b1, W2)
  agg2p = _spmm_kernel(E, Npad, C)(zs, src3, dst3, ew2)
  return _final_call(N, C, Npad, Rb)(agg2p, zs, dinv, b2)


# speedup vs baseline: 32.7406x; 1.1922x over previous
import functools

import jax
import jax.numpy as jnp
from jax import lax
from jax.experimental import pallas as pl
from jax.experimental.pallas import tpu as pltpu
from jax.experimental.pallas import tpu_sc as plsc

NC = 2
NS = 16
NW = NC * NS
L = 16
CH = 40

_mesh = lambda: plsc.VectorSubcoreMesh(
    core_axis_name="c", subcore_axis_name="s", num_cores=NC, num_subcores=NS)


@functools.lru_cache(maxsize=None)
def _deg_kernel(E, Npad):
  EPT = E // NW
  NCH = EPT // CH

  @functools.partial(
      pl.kernel, mesh=_mesh(),
      compiler_params=pltpu.CompilerParams(needs_layout_passes=False, use_tc_tiling_on_sc=False),
      out_type=jax.ShapeDtypeStruct((NW, Npad), jnp.float32),
      scratch_types=[
          pltpu.VMEM((Npad,), jnp.float32),
          pltpu.VMEM((EPT,), jnp.int32),
          pltpu.VMEM((EPT,), jnp.float32),
      ],
  )
  def k(dst_hbm, ew_hbm, out_hbm, acc, dstv, ewv):
    cid = lax.axis_index("c")
    sid = lax.axis_index("s")
    wid = sid * NC + cid
    zv = jnp.zeros((L,), jnp.float32)

    def zero(i, _):
      acc[pl.ds(i * L, L)] = zv
      return 0
    lax.fori_loop(0, Npad // L, zero, 0)

    pltpu.sync_copy(dst_hbm.at[wid], dstv)
    pltpu.sync_copy(ew_hbm.at[wid], ewv)

    def step(i, _):
      idx = dstv[pl.ds(i * L, L)]
      w = ewv[pl.ds(i * L, L)]
      plsc.addupdate_scatter(acc, [idx], w)
      return 0
    lax.fori_loop(0, EPT // L, step, 0)
    pltpu.sync_copy(acc, out_hbm.at[wid])

  return k


@functools.lru_cache(maxsize=None)
def _spmm_kernel(E, Npad, D, CH, NBUF):
  EPT = E // NW
  NCH = EPT // CH
  SEG = Npad // NS
  NZ = SEG // CH

  NO = -(-NCH // NBUF)

  @functools.partial(
      pl.kernel, mesh=_mesh(),
      compiler_params=pltpu.CompilerParams(needs_layout_passes=False, use_tc_tiling_on_sc=False),
      out_type=jax.ShapeDtypeStruct((NC, Npad, D), jnp.float32),
      scratch_types=[
          pltpu.VMEM_SHARED((Npad, D), jnp.float32),
          pltpu.VMEM((NBUF, CH, D), jnp.float32),
          pltpu.VMEM((NCH, CH), jnp.int32),
          pltpu.VMEM((NCH, CH), jnp.int32),
          pltpu.VMEM((EPT,), jnp.float32),
          pltpu.SemaphoreType.DMA((NBUF,)),
          pltpu.SemaphoreType.DMA((NBUF,)),
      ],
  )
  def k(tab_hbm, src_hbm, dst_hbm, ew_hbm, out_hbm,
        acc, rows, srcv, dstv, ewv, gsem, ssem):
    cid = lax.axis_index("c")
    sid = lax.axis_index("s")
    wid = sid * NC + cid
    zv = jnp.zeros((L,), jnp.float32)

    pltpu.sync_copy(src_hbm.at[wid], srcv)
    pltpu.sync_copy(dst_hbm.at[wid], dstv)
    pltpu.sync_copy(ew_hbm.at[wid], ewv)

    def zfill(i, _):
      for f in range(D // L):
        rows[0, i, pl.ds(f * L, L)] = zv
      return 0
    lax.fori_loop(0, CH, zfill, 0)

    def zcopy(i, _):
      pltpu.sync_copy(rows.at[0], acc.at[pl.ds(sid * SEG + i * CH, CH)])
      return 0
    lax.fori_loop(0, NZ, zcopy, 0)
    plsc.subcore_barrier()

    def gather_start(c, b):
      pltpu.async_copy(tab_hbm.at[srcv.at[c]], rows.at[b], gsem.at[b])

    for b in range(NBUF):
      gather_start(b, b)

    def outer(kk, _):
      for b in range(NBUF):
        c = kk * NBUF + b

        @pl.when(c < NCH)
        def _():
          pltpu.make_async_copy(
              tab_hbm.at[srcv.at[c]], rows.at[b], gsem.at[b]).wait()
          rb = rows.at[b]
          e0 = c * CH

          def scale(j, _):
            w = plsc.load_gather(ewv, [jnp.full((L,), e0, jnp.int32) + j])
            for f in range(D // L):
              rb[j, pl.ds(f * L, L)] = rb[j, pl.ds(f * L, L)] * w
            return 0
          lax.fori_loop(0, CH, scale, 0)
          pltpu.async_copy(rb, acc.at[dstv.at[c]], ssem.at[b], add=True)

      for b in range(NBUF):
        c = kk * NBUF + b

        @pl.when(c < NCH)
        def _():
          pltpu.make_async_copy(
              rows.at[b], acc.at[dstv.at[c]], ssem.at[b]).wait()

        nc = c + NBUF

        @pl.when(nc < NCH)
        def _():
          gather_start(nc, b)
      return 0
    lax.fori_loop(0, NO, outer, 0)

    plsc.subcore_barrier()
    pltpu.sync_copy(acc.at[pl.ds(sid * SEG, SEG)],
                    out_hbm.at[cid, pl.ds(sid * SEG, SEG)])

  return k


@functools.lru_cache(maxsize=None)
def _prep_call(N, F, Npad, Rb):
  def body(degp_ref, x_ref, dinv_ref, xs_ref):
    deg = jnp.sum(degp_ref[...], axis=0) + 1.0
    dinv = lax.rsqrt(deg)
    dinv_ref[...] = dinv
    xs_ref[...] = x_ref[...] * dinv[:, None]

  return pl.pallas_call(
      body,
      grid=(Npad // Rb,),
      in_specs=[
          pl.BlockSpec((NW, Rb), lambda i: (0, i)),
          pl.BlockSpec((Rb, F), lambda i: (i, 0)),
      ],
      out_specs=[
          pl.BlockSpec((Rb,), lambda i: (i,)),
          pl.BlockSpec((Rb, F), lambda i: (i, 0)),
      ],
      out_shape=[
          jax.ShapeDtypeStruct((N,), jnp.float32),
          jax.ShapeDtypeStruct((N, F), jnp.float32),
      ],
  )


@functools.lru_cache(maxsize=None)
def _mid_call(N, F, H, C, Npad, Rb):
  def body(aggp_ref, xs_ref, dinv_ref, w1_ref, b1_ref, w2_ref, zs_ref):
    dinv = dinv_ref[...]
    a = (aggp_ref[0] + aggp_ref[1] + xs_ref[...]) * dinv[:, None]
    h = jnp.dot(a, w1_ref[...], preferred_element_type=jnp.float32) + b1_ref[...]
    h = jnp.where(h > 0.0, h, jnp.exp(h) - 1.0)
    z = jnp.dot(h, w2_ref[...], preferred_element_type=jnp.float32)
    zs_ref[...] = z * dinv[:, None]

  return pl.pallas_call(
      body,
      grid=(Npad // Rb,),
      in_specs=[
          pl.BlockSpec((NC, Rb, F), lambda i: (0, i, 0)),
          pl.BlockSpec((Rb, F), lambda i: (i, 0)),
          pl.BlockSpec((Rb,), lambda i: (i,)),
          pl.BlockSpec((F, H), lambda i: (0, 0)),
          pl.BlockSpec((H,), lambda i: (0,)),
          pl.BlockSpec((H, C), lambda i: (0, 0)),
      ],
      out_specs=pl.BlockSpec((Rb, C), lambda i: (i, 0)),
      out_shape=jax.ShapeDtypeStruct((N, C), jnp.float32),
  )


@functools.lru_cache(maxsize=None)
def _final_call(N, C, Npad, Rb):
  def body(agg2p_ref, zs_ref, dinv_ref, b2_ref, out_ref):
    dinv = dinv_ref[...]
    o = (agg2p_ref[0] + agg2p_ref[1] + zs_ref[...]) * dinv[:, None] + b2_ref[...]
    m = jnp.max(o, axis=1, keepdims=True)
    e = jnp.exp(o - m)
    s = jnp.sum(e, axis=1, keepdims=True)
    out_ref[...] = o - m - jnp.log(s)

  return pl.pallas_call(
      body,
      grid=(Npad // Rb,),
      in_specs=[
          pl.BlockSpec((NC, Rb, C), lambda i: (0, i, 0)),
          pl.BlockSpec((Rb, C), lambda i: (i, 0)),
          pl.BlockSpec((Rb,), lambda i: (i,)),
          pl.BlockSpec((C,), lambda i: (0,)),
      ],
      out_specs=pl.BlockSpec((Rb, C), lambda i: (i, 0)),
      out_shape=jax.ShapeDtypeStruct((N, C), jnp.float32),
  )


def kernel(x, edge_index, edge_weight, W1, b1, W2, b2):
  N, F = x.shape
  E = edge_weight.shape[0]
  H = W1.shape[1]
  C = W2.shape[1]
  assert E % (NW * CH) == 0

  seg = ((N + NS - 1) // NS + CH - 1) // CH * CH
  Npad = NS * seg
  Rb = 512
  assert Npad % Rb == 0

  EPT = E // NW
  CH1, NB1 = 40, 3
  CH2, NB2 = 80, 5
  src1 = edge_index[0].reshape(NW, EPT // CH1, CH1)
  dst1 = edge_index[1].reshape(NW, EPT // CH1, CH1)
  src2 = edge_index[0].reshape(NW, EPT // CH2, CH2)
  dst2r = edge_index[1].reshape(NW, EPT // CH2, CH2)
  dst2 = edge_index[1].reshape(NW, EPT)
  ew2 = edge_weight.reshape(NW, EPT)

  degp = _deg_kernel(E, Npad)(dst2, ew2)
  dinv, xs = _prep_call(N, F, Npad, Rb)(degp, x)
  aggp = _spmm_kernel(E, Npad, F, CH1, NB1)(xs, src1, dst1, ew2)
  zs = _mid_call(N, F, H, C, Npad, Rb)(aggp, xs, dinv, W1, b1, W2)
  agg2p = _spmm_kernel(E, Npad, C, CH2, NB2)(zs, src2, dst2r, ew2)
  return _final_call(N, C, Npad, Rb)(agg2p, zs, dinv, b2)
